# Initial kernel scaffold; baseline (speedup 1.0000x reference)
import functools

import jax
import jax.numpy as jnp
from jax import lax
from jax.experimental import pallas as pl
from jax.experimental.pallas import tpu as pltpu
from jax.experimental.pallas import tpu_sc as plsc

Q = 1024
S = 100000
D = 256
NCLS = 64
K = 32
LAM = 0.5
CH = 128
SB = 2048
S_PAD = 100352
NBLK = S_PAD // SB
NC = S_PAD // CH
QT = 128
NEG = -3.0e38


def _stage1_body(q_ref, s_ref, l_ref, scores_ref, cmax_ref, ps_ref, smx_ref,
                 pconf_ref, ppred_ref, acc_ref, cnt_ref):
    b = pl.program_id(0)
    x = q_ref[...]
    sblk = s_ref[...]
    labels = l_ref[0, 0, :]

    cls = lax.broadcasted_iota(jnp.int32, (NCLS, SB), 0)
    onehot_t = (labels[None, :] == cls).astype(jnp.float32)
    pacc = lax.dot_general(onehot_t, sblk, (((1,), (0,)), ((), ())),
                           preferred_element_type=jnp.float32)

    @pl.when(b == 0)
    def _init():
        acc_ref[...] = jnp.zeros_like(acc_ref)
        cnt_ref[...] = jnp.zeros_like(cnt_ref)

    acc_ref[...] += pacc
    cnt_ref[...] += jnp.sum(onehot_t, axis=1, keepdims=True)

    q_sq = jnp.sum(x * x, axis=1, keepdims=True)
    s_sq = jnp.sum(sblk * sblk, axis=1)[None, :]
    qs = lax.dot_general(x, sblk, (((1,), (1,)), ((), ())),
                         preferred_element_type=jnp.float32)
    scores = -((q_sq - 2.0 * qs) + s_sq)
    col = lax.broadcasted_iota(jnp.int32, (Q, SB), 1) + b * SB
    scores = jnp.where(col < S, scores, NEG)
    scores_ref[...] = scores
    cmax_ref[...] = jnp.max(scores.reshape(Q, SB // CH, CH), axis=2)

    @pl.when(b == NBLK - 1)
    def _final():
        counts = jnp.maximum(cnt_ref[...], 1.0)
        protos = acc_ref[...] / counts
        p_sq = jnp.sum(protos * protos, axis=1)[None, :]
        qp = lax.dot_general(x, protos, (((1,), (1,)), ((), ())),
                             preferred_element_type=jnp.float32)
        pscores = -((q_sq - 2.0 * qp) + p_sq)
        ps_ref[...] = pscores
        m = jnp.max(pscores, axis=1, keepdims=True)
        e = jnp.exp(pscores - m)
        z = jnp.sum(e, axis=1, keepdims=True)
        smx = e / z
        smx_ref[...] = smx
        pconf_ref[...] = jnp.max(smx, axis=1, keepdims=True)
        ci = lax.broadcasted_iota(jnp.int32, (Q, NCLS), 1)
        ppred_ref[...] = jnp.min(
            jnp.where(pscores == m, ci, NCLS), axis=1, keepdims=True)


def _stage1(qf, sf, labels_pad):
    out_shapes = (
        jax.ShapeDtypeStruct((Q, S_PAD), jnp.float32),
        jax.ShapeDtypeStruct((Q, NC), jnp.float32),
        jax.ShapeDtypeStruct((Q, NCLS), jnp.float32),
        jax.ShapeDtypeStruct((Q, NCLS), jnp.float32),
        jax.ShapeDtypeStruct((Q, 1), jnp.float32),
        jax.ShapeDtypeStruct((Q, 1), jnp.int32),
    )
    return pl.pallas_call(
        _stage1_body,
        grid=(NBLK,),
        in_specs=[
            pl.BlockSpec((Q, D), lambda b: (0, 0)),
            pl.BlockSpec((SB, D), lambda b: (b, 0)),
            pl.BlockSpec((1, 1, SB), lambda b: (b, 0, 0)),
        ],
        out_specs=(
            pl.BlockSpec((Q, SB), lambda b: (0, b)),
            pl.BlockSpec((Q, SB // CH), lambda b: (0, b)),
            pl.BlockSpec((Q, NCLS), lambda b: (0, 0)),
            pl.BlockSpec((Q, NCLS), lambda b: (0, 0)),
            pl.BlockSpec((Q, 1), lambda b: (0, 0)),
            pl.BlockSpec((Q, 1), lambda b: (0, 0)),
        ),
        out_shape=out_shapes,
        scratch_shapes=[
            pltpu.VMEM((NCLS, D), jnp.float32),
            pltpu.VMEM((NCLS, 1), jnp.float32),
        ],
    )(qf, sf, labels_pad)


def _stage2_body(cm_ref, rows_ref, x_ref, cand_ref):
    t = pl.program_id(0)
    ci = lax.broadcasted_iota(jnp.int32, (QT, NC), 1)
    x_ref[...] = cm_ref[...]
    ki = lax.broadcasted_iota(jnp.int32, (QT, K), 1)

    def pick(k, _):
        xx = x_ref[...]
        m = jnp.max(xx, axis=1, keepdims=True)
        cid = jnp.min(jnp.where(xx == m, ci, NC), axis=1, keepdims=True)
        x_ref[...] = jnp.where(ci == cid, NEG, xx)
        cand_ref[...] = jnp.where(ki == k, cid, cand_ref[...])
        return 0

    lax.fori_loop(0, K, pick, 0, unroll=False)

    qi = lax.broadcasted_iota(jnp.int32, (QT, K), 0) + t * QT

    def srt(k, _):
        cc = cand_ref[...]
        mn = jnp.min(cc, axis=1, keepdims=True)
        cand_ref[...] = jnp.where(cc == mn, NC + 1, cc)
        rows_ref[...] = jnp.where(ki == k, qi * NC + mn, rows_ref[...])
        return 0

    lax.fori_loop(0, K, srt, 0, unroll=False)


def _stage2(cmax):
    return pl.pallas_call(
        _stage2_body,
        grid=(Q // QT,),
        in_specs=[pl.BlockSpec((QT, NC), lambda t: (t, 0))],
        out_specs=pl.BlockSpec((QT, K), lambda t: (t, 0)),
        out_shape=jax.ShapeDtypeStruct((Q, K), jnp.int32),
        scratch_shapes=[
            pltpu.VMEM((QT, NC), jnp.float32),
            pltpu.VMEM((QT, K), jnp.int32),
        ],
    )(cmax)


def kernel(query_features, support_features, support_labels):
    qf = query_features.astype(jnp.float32)
    sf = support_features.astype(jnp.float32)
    labels_i32 = support_labels.astype(jnp.int32)

    sf_pad = jnp.concatenate(
        [sf, jnp.zeros((S_PAD - S, D), jnp.float32)], axis=0)
    labels_pad = jnp.concatenate(
        [labels_i32, jnp.full((S_PAD - S,), NCLS, jnp.int32)]).reshape(
            NBLK, 1, SB)

    scores, cmax, pscores, smx, pconf, ppred = _stage1(qf, sf_pad, labels_pad)
    rows = _stage2(cmax)

    indices, kdist, knn_pred, new_pred, fused = _stage3_sc(
        scores.reshape(Q * NC, CH), rows, labels_i32, smx,
        pconf.reshape(Q), ppred.reshape(Q))

    return (pscores, indices, kdist,
            knn_pred.astype(support_labels.dtype),
            new_pred.astype(support_labels.dtype), fused)


NW = 32
QW = Q // NW


def _stage3_sc(scores2d, rows, labels_i32, smx, pconf, ppred):
    c = rows % NC
    cand = scores2d[rows]
    gidx = c[:, :, None] * CH + jnp.arange(CH)[None, None, :]
    flat = cand.reshape(Q, K * CH)
    gflat = gidx.reshape(Q, K * CH)
    neg_top, pos = lax.top_k(flat, K)
    indices = jnp.take_along_axis(gflat, pos, axis=1).astype(jnp.int32)
    kdist = -neg_top
    lab = labels_i32[indices]
    oh = jax.nn.one_hot(lab, NCLS, dtype=jnp.float32)
    votes = jnp.sum(oh, axis=1)
    knn_pred = jnp.argmax(votes, axis=1).astype(jnp.int32)
    w = jax.nn.softmax(neg_top, axis=1)
    soft = jnp.sum(w[:, :, None] * oh, axis=1)
    knn_conf = jnp.max(votes, axis=1) / float(K)
    new_pred = jnp.where(LAM * pconf > (1.0 - LAM) * knn_conf,
                         ppred, knn_pred).astype(jnp.int32)
    fused = LAM * smx + (1.0 - LAM) * soft
    return indices, kdist, knn_pred, new_pred, fused


# speedup vs baseline: 8.7724x; 8.7724x over previous
import functools

import jax
import jax.numpy as jnp
from jax import lax
from jax.experimental import pallas as pl
from jax.experimental.pallas import tpu as pltpu
from jax.experimental.pallas import tpu_sc as plsc

Q = 1024
S = 100000
D = 256
NCLS = 64
K = 32
LAM = 0.5
CH = 128
SB = 2048
S_PAD = 100352
NBLK = S_PAD // SB
NC = S_PAD // CH
QT = 128
NEG = -3.0e38


def _i0():
    return jnp.int32(0)


def _stage1_body(q_ref, s_ref, l_ref, scores_ref, cmax_ref, ps_ref, smx_ref,
                 pconf_ref, ppred_ref, acc_ref, cnt_ref):
    b = pl.program_id(0)
    x = q_ref[...]
    sblk = s_ref[...]
    labels = l_ref[0, 0, :]

    cls = lax.broadcasted_iota(jnp.int32, (NCLS, SB), 0)
    onehot_t = (labels[None, :] == cls).astype(jnp.float32)
    pacc = lax.dot_general(onehot_t, sblk, (((1,), (0,)), ((), ())),
                           preferred_element_type=jnp.float32)

    @pl.when(b == 0)
    def _init():
        acc_ref[...] = jnp.zeros_like(acc_ref)
        cnt_ref[...] = jnp.zeros_like(cnt_ref)

    acc_ref[...] += pacc
    cnt_ref[...] += jnp.sum(onehot_t, axis=1, keepdims=True)

    q_sq = jnp.sum(x * x, axis=1, keepdims=True)
    s_sq = jnp.sum(sblk * sblk, axis=1)[None, :]
    qs = lax.dot_general(x, sblk, (((1,), (1,)), ((), ())),
                         preferred_element_type=jnp.float32)
    scores = -((q_sq - jnp.float32(2.0) * qs) + s_sq)
    col = lax.broadcasted_iota(jnp.int32, (Q, SB), 1) + b * SB
    scores = jnp.where(col < S, scores, jnp.float32(NEG))
    scores_ref[...] = scores
    cmax_ref[0] = jnp.max(scores.reshape(Q, SB // CH, CH), axis=2)

    @pl.when(b == NBLK - 1)
    def _final():
        counts = jnp.maximum(cnt_ref[...], jnp.float32(1.0))
        protos = acc_ref[...] / counts
        p_sq = jnp.sum(protos * protos, axis=1)[None, :]
        qp = lax.dot_general(x, protos, (((1,), (1,)), ((), ())),
                             preferred_element_type=jnp.float32)
        pscores = -((q_sq - jnp.float32(2.0) * qp) + p_sq)
        ps_ref[...] = pscores
        m = jnp.max(pscores, axis=1, keepdims=True)
        e = jnp.exp(pscores - m)
        z = jnp.sum(e, axis=1, keepdims=True)
        smx = e / z
        smx_ref[...] = smx
        pconf_ref[...] = jnp.max(smx, axis=1, keepdims=True)
        ci = lax.broadcasted_iota(jnp.int32, (Q, NCLS), 1)
        ppred_ref[...] = jnp.min(
            jnp.where(pscores == m, ci, NCLS), axis=1, keepdims=True)


def _stage1(qf, sf, labels_pad):
    out_shapes = (
        jax.ShapeDtypeStruct((Q, S_PAD), jnp.float32),
        jax.ShapeDtypeStruct((NBLK, Q, SB // CH), jnp.float32),
        jax.ShapeDtypeStruct((Q, NCLS), jnp.float32),
        jax.ShapeDtypeStruct((Q, NCLS), jnp.float32),
        jax.ShapeDtypeStruct((Q, 1), jnp.float32),
        jax.ShapeDtypeStruct((Q, 1), jnp.int32),
    )
    return pl.pallas_call(
        _stage1_body,
        grid=(NBLK,),
        in_specs=[
            pl.BlockSpec((Q, D), lambda b: (_i0(), _i0())),
            pl.BlockSpec((SB, D), lambda b: (b, _i0())),
            pl.BlockSpec((1, 1, SB), lambda b: (b, _i0(), _i0())),
        ],
        out_specs=(
            pl.BlockSpec((Q, SB), lambda b: (_i0(), b)),
            pl.BlockSpec((1, Q, SB // CH), lambda b: (b, _i0(), _i0())),
            pl.BlockSpec((Q, NCLS), lambda b: (_i0(), _i0())),
            pl.BlockSpec((Q, NCLS), lambda b: (_i0(), _i0())),
            pl.BlockSpec((Q, 1), lambda b: (_i0(), _i0())),
            pl.BlockSpec((Q, 1), lambda b: (_i0(), _i0())),
        ),
        out_shape=out_shapes,
        scratch_shapes=[
            pltpu.VMEM((NCLS, D), jnp.float32),
            pltpu.VMEM((NCLS, 1), jnp.float32),
        ],
    )(qf, sf, labels_pad)


def _stage2_body(cm_ref, rows_ref, x_ref, cand_ref):
    t = pl.program_id(0)
    ci = lax.broadcasted_iota(jnp.int32, (QT, NC), 1)
    x_ref[...] = cm_ref[...]
    ki = lax.broadcasted_iota(jnp.int32, (QT, K), 1)

    def pick(_, c):
        xx = x_ref[...]
        m = jnp.max(xx, axis=1, keepdims=True)
        cid = jnp.min(jnp.where(xx == m, ci, NC), axis=1, keepdims=True)
        x_ref[...] = jnp.where(ci == cid, jnp.float32(NEG), xx)
        cand_ref[...] = jnp.where(ki == c, cid, cand_ref[...])
        return c + 1

    lax.fori_loop(0, K, pick, jnp.int32(0), unroll=False)

    qi = lax.broadcasted_iota(jnp.int32, (QT, K), 0) + t * QT

    def srt(_, c):
        cc = cand_ref[...]
        mn = jnp.min(cc, axis=1, keepdims=True)
        cand_ref[...] = jnp.where(cc == mn, NC + 1, cc)
        rows_ref[...] = jnp.where(ki == c, qi * NC + mn, rows_ref[...])
        return c + 1

    lax.fori_loop(0, K, srt, jnp.int32(0), unroll=False)


def _stage2(cmax):
    return pl.pallas_call(
        _stage2_body,
        grid=(Q // QT,),
        in_specs=[pl.BlockSpec((QT, NC), lambda t: (t, _i0()))],
        out_specs=pl.BlockSpec((QT, K), lambda t: (t, _i0())),
        out_shape=jax.ShapeDtypeStruct((Q, K), jnp.int32),
        scratch_shapes=[
            pltpu.VMEM((QT, NC), jnp.float32),
            pltpu.VMEM((QT, K), jnp.int32),
        ],
    )(cmax)


def kernel(query_features, support_features, support_labels):
    qf = query_features.astype(jnp.float32)
    sf = support_features.astype(jnp.float32)
    labels_i32 = support_labels.astype(jnp.int32)

    sf_pad = jnp.concatenate(
        [sf, jnp.zeros((S_PAD - S, D), jnp.float32)], axis=0)
    labels_pad = jnp.concatenate(
        [labels_i32, jnp.full((S_PAD - S,), NCLS, jnp.int32)]).reshape(
            NBLK, 1, SB)

    scores, cmax_raw, pscores, smx, pconf, ppred = _stage1(qf, sf_pad,
                                                           labels_pad)
    cmax = cmax_raw.transpose(1, 0, 2).reshape(Q, NC)
    rows = _stage2(cmax)

    indices, kdist, knn_pred, new_pred, fused = _stage3_sc(
        scores.reshape(Q * NC, CH), rows, labels_i32, smx,
        pconf.reshape(Q), ppred.reshape(Q))

    return (pscores, indices, kdist,
            knn_pred.astype(support_labels.dtype),
            new_pred.astype(support_labels.dtype), fused)


NW = 32
QW = Q // NW
NG = 16


def _sc_body(scores_hbm, rows_hbm, labels_hbm, smx_hbm, pconf_hbm, ppred_hbm,
             idx_out, kd_out, kp_out, np_out, fu_out,
             labels_v, rows_v, cand_v, labsel_v, w_v,
             oidx_v, okd_v, okp_v, onp_v, ofu_v, smx_v, pconf_v, ppred_v,
             sem):
    wid = lax.axis_index("s") * 2 + lax.axis_index("c")
    q0 = (wid * QW).astype(jnp.int32)
    lane = lax.broadcasted_iota(jnp.int32, (16,), 0)
    izeros = jnp.zeros((16,), jnp.int32)
    lane0 = lane == 0

    pltpu.sync_copy(labels_hbm, labels_v)
    pltpu.sync_copy(rows_hbm.at[pl.ds(q0, QW)], rows_v)
    pltpu.sync_copy(smx_hbm.at[pl.ds(q0, QW)], smx_v)
    pltpu.sync_copy(pconf_hbm.at[pl.ds(q0, QW)], pconf_v)
    pltpu.sync_copy(ppred_hbm.at[pl.ds(q0, QW)], ppred_v)

    def gmax_one(g):
        gm = jnp.full((16,), NEG, jnp.float32)
        for t in range(NG):
            row = 2 * g + t // 8
            vt = cand_v[row, pl.ds((t % 8) * 16, 16)]
            gm = jnp.maximum(gm, vt)
        return jnp.max(gm)

    def per_query(_, carry):
        i, kp0, kp1, np0, np1 = carry
        q = q0 + i
        pltpu.async_copy(scores_hbm.at[rows_v.at[i]], cand_v, sem).wait()

        def build(_, bcarry):
            g, M = bcarry
            return g + 1, jnp.where(lane == g, gmax_one(g), M)

        _, M = lax.fori_loop(
            0, NG, build,
            (jnp.int32(0), jnp.full((16,), NEG, jnp.float32)))

        def ext(_, ecarry):
            k, M, sv0, sv1, iv0, iv1 = ecarry
            m = jnp.max(M)
            g = jnp.min(jnp.where(M == m, lane, NG))
            p = jnp.int32(4096)
            for t in range(NG):
                row = 2 * g + t // 8
                vt = cand_v[row, pl.ds((t % 8) * 16, 16)]
                p = jnp.minimum(
                    p, jnp.min(jnp.where(vt == m, t * 16 + lane, 4096)))
            f = g * 256 + p
            row_f = lax.shift_right_logical(f, jnp.int32(7))
            col_f = jnp.bitwise_and(f, jnp.int32(127))
            plsc.store_scatter(cand_v, [izeros + row_f, izeros + col_f],
                               jnp.full((16,), NEG, jnp.float32), mask=lane0)
            M = jnp.where(lane == g, gmax_one(g), M)
            r_val = plsc.load_gather(
                rows_v, [izeros + i, izeros + row_f])[0]
            sup = (r_val - q * NC) * CH + col_f
            sv0 = jnp.where(lane == k, m, sv0)
            sv1 = jnp.where(lane == k - 16, m, sv1)
            iv0 = jnp.where(lane == k, sup, iv0)
            iv1 = jnp.where(lane == k - 16, sup, iv1)
            return k + 1, M, sv0, sv1, iv0, iv1

        fneg = jnp.full((16,), NEG, jnp.float32)
        _, M, sv0, sv1, iv0, iv1 = lax.fori_loop(
            0, K, ext, (jnp.int32(0), M, fneg, fneg, izeros, izeros))

        oidx_v[i, pl.ds(0, 16)] = iv0
        oidx_v[i, pl.ds(16, 16)] = iv1
        okd_v[i, pl.ds(0, 16)] = -sv0
        okd_v[i, pl.ds(16, 16)] = -sv1

        m0 = jnp.max(sv0)
        e0 = jnp.exp(sv0 - m0)
        e1 = jnp.exp(sv1 - m0)
        z = jnp.sum(e0) + jnp.sum(e1)
        w_v[pl.ds(0, 16)] = e0 / z
        w_v[pl.ds(16, 16)] = e1 / z

        labsel_v[pl.ds(0, 16)] = plsc.load_gather(labels_v, [iv0])
        labsel_v[pl.ds(16, 16)] = plsc.load_gather(labels_v, [iv1])

        cls = [lane + 16 * b for b in range(4)]

        def vote(_, vcarry):
            k = vcarry[0]
            v = list(vcarry[1:5])
            s = list(vcarry[5:])
            lab = plsc.load_gather(labsel_v, [izeros + k])[0]
            wk = plsc.load_gather(w_v, [izeros + k])[0]
            one = jnp.float32(1.0)
            zero = jnp.float32(0.0)
            for b in range(4):
                eq = cls[b] == lab
                v[b] = v[b] + jnp.where(eq, one, zero)
                s[b] = s[b] + jnp.where(eq, wk, zero)
            return (k + 1,) + tuple(v) + tuple(s)

        zeros = tuple(jnp.zeros((16,), jnp.float32) for _ in range(8))
        vs = lax.fori_loop(0, K, vote, (jnp.int32(0),) + zeros)
        votes = vs[1:5]
        soft = vs[5:]

        best_v = jnp.float32(-1.0)
        best_c = jnp.int32(NCLS)
        for b in range(4):
            bm = jnp.max(votes[b])
            pos = jnp.min(jnp.where(votes[b] == bm, cls[b], NCLS))
            take = bm > best_v
            best_c = jnp.where(take, pos, best_c)
            best_v = jnp.where(take, bm, best_v)
        knn_conf = best_v * jnp.float32(1.0 / K)
        npred = jnp.where(
            jnp.float32(LAM) * plsc.load_gather(pconf_v, [izeros + i])[0]
            > jnp.float32(1.0 - LAM) * knn_conf,
            plsc.load_gather(ppred_v, [izeros + i])[0], best_c)
        kp0 = jnp.where(lane == i, best_c, kp0)
        kp1 = jnp.where(lane == i - 16, best_c, kp1)
        np0 = jnp.where(lane == i, npred, np0)
        np1 = jnp.where(lane == i - 16, npred, np1)
        half = jnp.float32(0.5)
        for b in range(4):
            ofu_v[i, pl.ds(16 * b, 16)] = (
                half * smx_v[i, pl.ds(16 * b, 16)] + half * soft[b])
        return i + 1, kp0, kp1, np0, np1

    _, kp0, kp1, np0, np1 = lax.fori_loop(
        0, QW, per_query,
        (jnp.int32(0), izeros, izeros, izeros, izeros))
    okp_v[pl.ds(0, 16)] = kp0
    okp_v[pl.ds(16, 16)] = kp1
    onp_v[pl.ds(0, 16)] = np0
    onp_v[pl.ds(16, 16)] = np1

    pltpu.sync_copy(oidx_v, idx_out.at[pl.ds(q0, QW)])
    pltpu.sync_copy(okd_v, kd_out.at[pl.ds(q0, QW)])
    pltpu.sync_copy(okp_v, kp_out.at[pl.ds(q0, QW)])
    pltpu.sync_copy(onp_v, np_out.at[pl.ds(q0, QW)])
    pltpu.sync_copy(ofu_v, fu_out.at[pl.ds(q0, QW)])


def _stage3_sc(scores2d, rows, labels_i32, smx, pconf, ppred):
    mesh = plsc.VectorSubcoreMesh(core_axis_name="c", subcore_axis_name="s")
    f32 = jnp.float32
    i32 = jnp.int32
    sk = functools.partial(
        pl.kernel, mesh=mesh,
        compiler_params=pltpu.CompilerParams(needs_layout_passes=False),
        out_type=(
            jax.ShapeDtypeStruct((Q, K), i32),
            jax.ShapeDtypeStruct((Q, K), f32),
            jax.ShapeDtypeStruct((Q,), i32),
            jax.ShapeDtypeStruct((Q,), i32),
            jax.ShapeDtypeStruct((Q, NCLS), f32),
        ),
        scratch_types=[
            pltpu.VMEM((S,), i32),
            pltpu.VMEM((QW, K), i32),
            pltpu.VMEM((K, CH), f32),
            pltpu.VMEM((K,), i32),
            pltpu.VMEM((K,), f32),
            pltpu.VMEM((QW, K), i32),
            pltpu.VMEM((QW, K), f32),
            pltpu.VMEM((QW,), i32),
            pltpu.VMEM((QW,), i32),
            pltpu.VMEM((QW, NCLS), f32),
            pltpu.VMEM((QW, NCLS), f32),
            pltpu.VMEM((QW,), f32),
            pltpu.VMEM((QW,), i32),
            pltpu.SemaphoreType.DMA,
        ],
    )(_sc_body)
    return sk(scores2d, rows, labels_i32, smx, pconf, ppred)


# speedup vs baseline: 9.4775x; 1.0804x over previous
import functools

import jax
import jax.numpy as jnp
from jax import lax
from jax.experimental import pallas as pl
from jax.experimental.pallas import tpu as pltpu
from jax.experimental.pallas import tpu_sc as plsc

Q = 1024
S = 100000
D = 256
NCLS = 64
K = 32
LAM = 0.5
CH = 128
SB = 2048
S_PAD = 100352
NBLK = S_PAD // SB
NC = S_PAD // CH
QT = 128
NEG = -3.0e38


def _i0():
    return jnp.int32(0)


def _stage1_body(q_ref, s_ref, l_ref, scores_ref, cmax_ref, ps_ref, smx_ref,
                 pconf_ref, ppred_ref, acc_ref, cnt_ref, qsq_ref):
    b = pl.program_id(0)
    x = q_ref[...]
    sblk = s_ref[...]
    labels = l_ref[0, 0, :]

    cls = lax.broadcasted_iota(jnp.int32, (NCLS, SB), 0)
    onehot_t = (labels[None, :] == cls).astype(jnp.float32)
    pacc = lax.dot_general(onehot_t, sblk, (((1,), (0,)), ((), ())),
                           preferred_element_type=jnp.float32)

    @pl.when(b == 0)
    def _init():
        acc_ref[...] = jnp.zeros_like(acc_ref)
        cnt_ref[...] = jnp.zeros_like(cnt_ref)
        qsq_ref[...] = jnp.sum(x * x, axis=1, keepdims=True)

    acc_ref[...] += pacc
    cnt_ref[...] += jnp.sum(onehot_t, axis=1, keepdims=True)

    q_sq = qsq_ref[...]
    s_sq = jnp.sum(sblk * sblk, axis=1)[None, :]
    qs = lax.dot_general(x, sblk, (((1,), (1,)), ((), ())),
                         preferred_element_type=jnp.float32)
    scores = -((q_sq - jnp.float32(2.0) * qs) + s_sq)

    @pl.when(b < NBLK - 1)
    def _store_full():
        scores_ref[...] = scores
        cmax_ref[0] = jnp.max(scores.reshape(Q, SB // CH, CH), axis=2)

    @pl.when(b == NBLK - 1)
    def _store_masked():
        col = lax.broadcasted_iota(jnp.int32, (Q, SB), 1) + b * SB
        sm = jnp.where(col < S, scores, jnp.float32(NEG))
        scores_ref[...] = sm
        cmax_ref[0] = jnp.max(sm.reshape(Q, SB // CH, CH), axis=2)

    @pl.when(b == NBLK - 1)
    def _final():
        counts = jnp.maximum(cnt_ref[...], jnp.float32(1.0))
        protos = acc_ref[...] / counts
        p_sq = jnp.sum(protos * protos, axis=1)[None, :]
        qp = lax.dot_general(x, protos, (((1,), (1,)), ((), ())),
                             preferred_element_type=jnp.float32)
        pscores = -((q_sq - jnp.float32(2.0) * qp) + p_sq)
        ps_ref[...] = pscores
        m = jnp.max(pscores, axis=1, keepdims=True)
        e = jnp.exp(pscores - m)
        z = jnp.sum(e, axis=1, keepdims=True)
        smx = e / z
        smx_ref[...] = smx
        pconf_ref[...] = jnp.max(smx, axis=1, keepdims=True)
        ci = lax.broadcasted_iota(jnp.int32, (Q, NCLS), 1)
        ppred_ref[...] = jnp.min(
            jnp.where(pscores == m, ci, NCLS), axis=1, keepdims=True)


def _stage1(qf, sf, labels_pad):
    out_shapes = (
        jax.ShapeDtypeStruct((Q, S_PAD), jnp.float32),
        jax.ShapeDtypeStruct((NBLK, Q, SB // CH), jnp.float32),
        jax.ShapeDtypeStruct((Q, NCLS), jnp.float32),
        jax.ShapeDtypeStruct((Q, NCLS), jnp.float32),
        jax.ShapeDtypeStruct((Q, 1), jnp.float32),
        jax.ShapeDtypeStruct((Q, 1), jnp.int32),
    )
    return pl.pallas_call(
        _stage1_body,
        grid=(NBLK,),
        in_specs=[
            pl.BlockSpec((Q, D), lambda b: (_i0(), _i0())),
            pl.BlockSpec((SB, D), lambda b: (b, _i0())),
            pl.BlockSpec((1, 1, SB), lambda b: (b, _i0(), _i0())),
        ],
        out_specs=(
            pl.BlockSpec((Q, SB), lambda b: (_i0(), b)),
            pl.BlockSpec((1, Q, SB // CH), lambda b: (b, _i0(), _i0())),
            pl.BlockSpec((Q, NCLS), lambda b: (_i0(), _i0())),
            pl.BlockSpec((Q, NCLS), lambda b: (_i0(), _i0())),
            pl.BlockSpec((Q, 1), lambda b: (_i0(), _i0())),
            pl.BlockSpec((Q, 1), lambda b: (_i0(), _i0())),
        ),
        out_shape=out_shapes,
        scratch_shapes=[
            pltpu.VMEM((NCLS, D), jnp.float32),
            pltpu.VMEM((NCLS, 1), jnp.float32),
            pltpu.VMEM((Q, 1), jnp.float32),
        ],
    )(qf, sf, labels_pad)


def _stage2_body(cm_ref, rows_ref, x_ref, cand_ref):
    t = pl.program_id(0)
    ci = lax.broadcasted_iota(jnp.int32, (QT, NC), 1)
    x_ref[...] = cm_ref[...]
    ki = lax.broadcasted_iota(jnp.int32, (QT, K), 1)

    def pick(_, c):
        xx = x_ref[...]
        m = jnp.max(xx, axis=1, keepdims=True)
        cid = jnp.min(jnp.where(xx == m, ci, NC), axis=1, keepdims=True)
        x_ref[...] = jnp.where(ci == cid, jnp.float32(NEG), xx)
        cand_ref[...] = jnp.where(ki == c, cid, cand_ref[...])
        return c + 1

    lax.fori_loop(0, K, pick, jnp.int32(0), unroll=False)

    qi = lax.broadcasted_iota(jnp.int32, (QT, K), 0) + t * QT

    def srt(_, c):
        cc = cand_ref[...]
        mn = jnp.min(cc, axis=1, keepdims=True)
        cand_ref[...] = jnp.where(cc == mn, NC + 1, cc)
        rows_ref[...] = jnp.where(ki == c, qi * NC + mn, rows_ref[...])
        return c + 1

    lax.fori_loop(0, K, srt, jnp.int32(0), unroll=False)


def _stage2(cmax):
    return pl.pallas_call(
        _stage2_body,
        grid=(Q // QT,),
        in_specs=[pl.BlockSpec((QT, NC), lambda t: (t, _i0()))],
        out_specs=pl.BlockSpec((QT, K), lambda t: (t, _i0())),
        out_shape=jax.ShapeDtypeStruct((Q, K), jnp.int32),
        scratch_shapes=[
            pltpu.VMEM((QT, NC), jnp.float32),
            pltpu.VMEM((QT, K), jnp.int32),
        ],
    )(cmax)


def kernel(query_features, support_features, support_labels):
    qf = query_features.astype(jnp.float32)
    sf = support_features.astype(jnp.float32)
    labels_i32 = support_labels.astype(jnp.int32)

    labels_pad = jnp.concatenate(
        [labels_i32, jnp.full((S_PAD - S,), NCLS, jnp.int32)]).reshape(
            NBLK, 1, SB)

    scores, cmax_raw, pscores, smx, pconf, ppred = _stage1(qf, sf,
                                                           labels_pad)
    cmax = cmax_raw.transpose(1, 0, 2).reshape(Q, NC)
    rows = _stage2(cmax)

    indices, kdist, knn_pred, new_pred, fused = _stage3_sc(
        scores.reshape(Q * NC, CH), rows, labels_i32, smx,
        pconf.reshape(Q), ppred.reshape(Q))

    return (pscores, indices, kdist,
            knn_pred.astype(support_labels.dtype),
            new_pred.astype(support_labels.dtype), fused)


NW = 32
QW = Q // NW
NG = 16


def _sc_body(scores_hbm, rows_hbm, labels_hbm, smx_hbm, pconf_hbm, ppred_hbm,
             idx_out, kd_out, kp_out, np_out, fu_out,
             labels_v, rows_v, cand_v, labsel_v, w_v,
             oidx_v, okd_v, okp_v, onp_v, ofu_v, smx_v, pconf_v, ppred_v,
             sem):
    wid = lax.axis_index("s") * 2 + lax.axis_index("c")
    q0 = (wid * QW).astype(jnp.int32)
    lane = lax.broadcasted_iota(jnp.int32, (16,), 0)
    izeros = jnp.zeros((16,), jnp.int32)
    lane0 = lane == 0

    pltpu.sync_copy(labels_hbm, labels_v)
    pltpu.sync_copy(rows_hbm.at[pl.ds(q0, QW)], rows_v)
    pltpu.sync_copy(smx_hbm.at[pl.ds(q0, QW)], smx_v)
    pltpu.sync_copy(pconf_hbm.at[pl.ds(q0, QW)], pconf_v)
    pltpu.sync_copy(ppred_hbm.at[pl.ds(q0, QW)], ppred_v)

    def gmax_one(g):
        gm = jnp.full((16,), NEG, jnp.float32)
        for t in range(NG):
            row = 2 * g + t // 8
            vt = cand_v[row, pl.ds((t % 8) * 16, 16)]
            gm = jnp.maximum(gm, vt)
        return jnp.max(gm)

    def per_query(_, carry):
        i, kp0, kp1, np0, np1 = carry
        q = q0 + i
        pltpu.async_copy(scores_hbm.at[rows_v.at[i]], cand_v, sem).wait()

        def build(_, bcarry):
            g, M = bcarry
            return g + 1, jnp.where(lane == g, gmax_one(g), M)

        _, M = lax.fori_loop(
            0, NG, build,
            (jnp.int32(0), jnp.full((16,), NEG, jnp.float32)))

        def ext(_, ecarry):
            k, M, sv0, sv1, iv0, iv1 = ecarry
            m = jnp.max(M)
            g = jnp.min(jnp.where(M == m, lane, NG))
            p = jnp.int32(4096)
            for t in range(NG):
                row = 2 * g + t // 8
                vt = cand_v[row, pl.ds((t % 8) * 16, 16)]
                p = jnp.minimum(
                    p, jnp.min(jnp.where(vt == m, t * 16 + lane, 4096)))
            f = g * 256 + p
            row_f = lax.shift_right_logical(f, jnp.int32(7))
            col_f = jnp.bitwise_and(f, jnp.int32(127))
            plsc.store_scatter(cand_v, [izeros + row_f, izeros + col_f],
                               jnp.full((16,), NEG, jnp.float32), mask=lane0)
            M = jnp.where(lane == g, gmax_one(g), M)
            r_val = plsc.load_gather(
                rows_v, [izeros + i, izeros + row_f])[0]
            sup = (r_val - q * NC) * CH + col_f
            sv0 = jnp.where(lane == k, m, sv0)
            sv1 = jnp.where(lane == k - 16, m, sv1)
            iv0 = jnp.where(lane == k, sup, iv0)
            iv1 = jnp.where(lane == k - 16, sup, iv1)
            return k + 1, M, sv0, sv1, iv0, iv1

        fneg = jnp.full((16,), NEG, jnp.float32)
        _, M, sv0, sv1, iv0, iv1 = lax.fori_loop(
            0, K, ext, (jnp.int32(0), M, fneg, fneg, izeros, izeros))

        oidx_v[i, pl.ds(0, 16)] = iv0
        oidx_v[i, pl.ds(16, 16)] = iv1
        okd_v[i, pl.ds(0, 16)] = -sv0
        okd_v[i, pl.ds(16, 16)] = -sv1

        m0 = jnp.max(sv0)
        e0 = jnp.exp(sv0 - m0)
        e1 = jnp.exp(sv1 - m0)
        z = jnp.sum(e0) + jnp.sum(e1)
        w_v[pl.ds(0, 16)] = e0 / z
        w_v[pl.ds(16, 16)] = e1 / z

        labsel_v[pl.ds(0, 16)] = plsc.load_gather(labels_v, [iv0])
        labsel_v[pl.ds(16, 16)] = plsc.load_gather(labels_v, [iv1])

        cls = [lane + 16 * b for b in range(4)]

        def vote(_, vcarry):
            k = vcarry[0]
            v = list(vcarry[1:5])
            s = list(vcarry[5:])
            lab = plsc.load_gather(labsel_v, [izeros + k])[0]
            wk = plsc.load_gather(w_v, [izeros + k])[0]
            one = jnp.float32(1.0)
            zero = jnp.float32(0.0)
            for b in range(4):
                eq = cls[b] == lab
                v[b] = v[b] + jnp.where(eq, one, zero)
                s[b] = s[b] + jnp.where(eq, wk, zero)
            return (k + 1,) + tuple(v) + tuple(s)

        zeros = tuple(jnp.zeros((16,), jnp.float32) for _ in range(8))
        vs = lax.fori_loop(0, K, vote, (jnp.int32(0),) + zeros)
        votes = vs[1:5]
        soft = vs[5:]

        best_v = jnp.float32(-1.0)
        best_c = jnp.int32(NCLS)
        for b in range(4):
            bm = jnp.max(votes[b])
            pos = jnp.min(jnp.where(votes[b] == bm, cls[b], NCLS))
            take = bm > best_v
            best_c = jnp.where(take, pos, best_c)
            best_v = jnp.where(take, bm, best_v)
        knn_conf = best_v * jnp.float32(1.0 / K)
        npred = jnp.where(
            jnp.float32(LAM) * plsc.load_gather(pconf_v, [izeros + i])[0]
            > jnp.float32(1.0 - LAM) * knn_conf,
            plsc.load_gather(ppred_v, [izeros + i])[0], best_c)
        kp0 = jnp.where(lane == i, best_c, kp0)
        kp1 = jnp.where(lane == i - 16, best_c, kp1)
        np0 = jnp.where(lane == i, npred, np0)
        np1 = jnp.where(lane == i - 16, npred, np1)
        half = jnp.float32(0.5)
        for b in range(4):
            ofu_v[i, pl.ds(16 * b, 16)] = (
                half * smx_v[i, pl.ds(16 * b, 16)] + half * soft[b])
        return i + 1, kp0, kp1, np0, np1

    _, kp0, kp1, np0, np1 = lax.fori_loop(
        0, QW, per_query,
        (jnp.int32(0), izeros, izeros, izeros, izeros))
    okp_v[pl.ds(0, 16)] = kp0
    okp_v[pl.ds(16, 16)] = kp1
    onp_v[pl.ds(0, 16)] = np0
    onp_v[pl.ds(16, 16)] = np1

    pltpu.sync_copy(oidx_v, idx_out.at[pl.ds(q0, QW)])
    pltpu.sync_copy(okd_v, kd_out.at[pl.ds(q0, QW)])
    pltpu.sync_copy(okp_v, kp_out.at[pl.ds(q0, QW)])
    pltpu.sync_copy(onp_v, np_out.at[pl.ds(q0, QW)])
    pltpu.sync_copy(ofu_v, fu_out.at[pl.ds(q0, QW)])


def _stage3_sc(scores2d, rows, labels_i32, smx, pconf, ppred):
    mesh = plsc.VectorSubcoreMesh(core_axis_name="c", subcore_axis_name="s")
    f32 = jnp.float32
    i32 = jnp.int32
    sk = functools.partial(
        pl.kernel, mesh=mesh,
        compiler_params=pltpu.CompilerParams(needs_layout_passes=False),
        out_type=(
            jax.ShapeDtypeStruct((Q, K), i32),
            jax.ShapeDtypeStruct((Q, K), f32),
            jax.ShapeDtypeStruct((Q,), i32),
            jax.ShapeDtypeStruct((Q,), i32),
            jax.ShapeDtypeStruct((Q, NCLS), f32),
        ),
        scratch_types=[
            pltpu.VMEM((S,), i32),
            pltpu.VMEM((QW, K), i32),
            pltpu.VMEM((K, CH), f32),
            pltpu.VMEM((K,), i32),
            pltpu.VMEM((K,), f32),
            pltpu.VMEM((QW, K), i32),
            pltpu.VMEM((QW, K), f32),
            pltpu.VMEM((QW,), i32),
            pltpu.VMEM((QW,), i32),
            pltpu.VMEM((QW, NCLS), f32),
            pltpu.VMEM((QW, NCLS), f32),
            pltpu.VMEM((QW,), f32),
            pltpu.VMEM((QW,), i32),
            pltpu.SemaphoreType.DMA,
        ],
    )(_sc_body)
    return sk(scores2d, rows, labels_i32, smx, pconf, ppred)


# speedup vs baseline: 9.7937x; 1.0334x over previous
import functools

import jax
import jax.numpy as jnp
from jax import lax
from jax.experimental import pallas as pl
from jax.experimental.pallas import tpu as pltpu
from jax.experimental.pallas import tpu_sc as plsc

Q = 1024
S = 100000
D = 256
NCLS = 64
K = 32
LAM = 0.5
CH = 128
SB = 2048
S_PAD = 100352
NBLK = S_PAD // SB
NC = S_PAD // CH
QT = 128
NEG = -3.0e38


def _i0():
    return jnp.int32(0)


def _stage1_body(q_ref, s_ref, l_ref, scores_ref, cmax_ref, ps_ref, smx_ref,
                 pconf_ref, ppred_ref, acc_ref, cnt_ref, qsq_ref, x2_ref):
    b = pl.program_id(0)
    x = q_ref[...]
    sblk = s_ref[...]
    labels = l_ref[0, 0, :]

    cls = lax.broadcasted_iota(jnp.int32, (NCLS, SB), 0)
    onehot_t = (labels[None, :] == cls).astype(jnp.float32)
    pacc = lax.dot_general(onehot_t, sblk, (((1,), (0,)), ((), ())),
                           preferred_element_type=jnp.float32)

    @pl.when(b == 0)
    def _init():
        acc_ref[...] = jnp.zeros_like(acc_ref)
        cnt_ref[...] = jnp.zeros_like(cnt_ref)
        qsq_ref[...] = jnp.sum(x * x, axis=1, keepdims=True)
        x2_ref[...] = x + x

    acc_ref[...] += pacc
    cnt_ref[...] += jnp.sum(onehot_t, axis=1, keepdims=True)

    q_sq = qsq_ref[...]
    s_sq = jnp.sum(sblk * sblk, axis=1)[None, :]
    qs2 = lax.dot_general(x2_ref[...], sblk, (((1,), (1,)), ((), ())),
                          preferred_element_type=jnp.float32)
    scores = (qs2 - q_sq) - s_sq

    @pl.when(b < NBLK - 1)
    def _store_full():
        scores_ref[...] = scores
        cmax_ref[0] = jnp.max(scores.reshape(Q, SB // CH, CH), axis=2)

    @pl.when(b == NBLK - 1)
    def _store_masked():
        col = lax.broadcasted_iota(jnp.int32, (Q, SB), 1) + b * SB
        sm = jnp.where(col < S, scores, jnp.float32(NEG))
        scores_ref[...] = sm
        cmax_ref[0] = jnp.max(sm.reshape(Q, SB // CH, CH), axis=2)

    @pl.when(b == NBLK - 1)
    def _final():
        counts = jnp.maximum(cnt_ref[...], jnp.float32(1.0))
        protos = acc_ref[...] / counts
        p_sq = jnp.sum(protos * protos, axis=1)[None, :]
        qp = lax.dot_general(x, protos, (((1,), (1,)), ((), ())),
                             preferred_element_type=jnp.float32)
        pscores = -((q_sq - jnp.float32(2.0) * qp) + p_sq)
        ps_ref[...] = pscores
        m = jnp.max(pscores, axis=1, keepdims=True)
        e = jnp.exp(pscores - m)
        z = jnp.sum(e, axis=1, keepdims=True)
        smx = e / z
        smx_ref[...] = smx
        pconf_ref[...] = jnp.max(smx, axis=1, keepdims=True)
        ci = lax.broadcasted_iota(jnp.int32, (Q, NCLS), 1)
        ppred_ref[...] = jnp.min(
            jnp.where(pscores == m, ci, NCLS), axis=1, keepdims=True)


def _stage1(qf, sf, labels_pad):
    out_shapes = (
        jax.ShapeDtypeStruct((Q, S_PAD), jnp.float32),
        jax.ShapeDtypeStruct((NBLK, Q, SB // CH), jnp.float32),
        jax.ShapeDtypeStruct((Q, NCLS), jnp.float32),
        jax.ShapeDtypeStruct((Q, NCLS), jnp.float32),
        jax.ShapeDtypeStruct((Q, 1), jnp.float32),
        jax.ShapeDtypeStruct((Q, 1), jnp.int32),
    )
    return pl.pallas_call(
        _stage1_body,
        grid=(NBLK,),
        in_specs=[
            pl.BlockSpec((Q, D), lambda b: (_i0(), _i0())),
            pl.BlockSpec((SB, D), lambda b: (b, _i0())),
            pl.BlockSpec((1, 1, SB), lambda b: (b, _i0(), _i0())),
        ],
        out_specs=(
            pl.BlockSpec((Q, SB), lambda b: (_i0(), b)),
            pl.BlockSpec((1, Q, SB // CH), lambda b: (b, _i0(), _i0())),
            pl.BlockSpec((Q, NCLS), lambda b: (_i0(), _i0())),
            pl.BlockSpec((Q, NCLS), lambda b: (_i0(), _i0())),
            pl.BlockSpec((Q, 1), lambda b: (_i0(), _i0())),
            pl.BlockSpec((Q, 1), lambda b: (_i0(), _i0())),
        ),
        out_shape=out_shapes,
        scratch_shapes=[
            pltpu.VMEM((NCLS, D), jnp.float32),
            pltpu.VMEM((NCLS, 1), jnp.float32),
            pltpu.VMEM((Q, 1), jnp.float32),
            pltpu.VMEM((Q, D), jnp.float32),
        ],
    )(qf, sf, labels_pad)


def _stage2_body(cm_ref, rows_ref, x_ref, cand_ref):
    t = pl.program_id(0)
    ci = lax.broadcasted_iota(jnp.int32, (QT, NC), 1)
    x_ref[...] = cm_ref[...]
    ki = lax.broadcasted_iota(jnp.int32, (QT, K), 1)

    def pick(_, c):
        xx = x_ref[...]
        m = jnp.max(xx, axis=1, keepdims=True)
        cid = jnp.min(jnp.where(xx == m, ci, NC), axis=1, keepdims=True)
        x_ref[...] = jnp.where(ci == cid, jnp.float32(NEG), xx)
        cand_ref[...] = jnp.where(ki == c, cid, cand_ref[...])
        return c + 1

    lax.fori_loop(0, K, pick, jnp.int32(0), unroll=False)

    qi = lax.broadcasted_iota(jnp.int32, (QT, K), 0) + t * QT

    def srt(_, c):
        cc = cand_ref[...]
        mn = jnp.min(cc, axis=1, keepdims=True)
        cand_ref[...] = jnp.where(cc == mn, NC + 1, cc)
        rows_ref[...] = jnp.where(ki == c, qi * NC + mn, rows_ref[...])
        return c + 1

    lax.fori_loop(0, K, srt, jnp.int32(0), unroll=False)


def _stage2(cmax):
    return pl.pallas_call(
        _stage2_body,
        grid=(Q // QT,),
        in_specs=[pl.BlockSpec((QT, NC), lambda t: (t, _i0()))],
        out_specs=pl.BlockSpec((QT, K), lambda t: (t, _i0())),
        out_shape=jax.ShapeDtypeStruct((Q, K), jnp.int32),
        scratch_shapes=[
            pltpu.VMEM((QT, NC), jnp.float32),
            pltpu.VMEM((QT, K), jnp.int32),
        ],
    )(cmax)


def kernel(query_features, support_features, support_labels):
    qf = query_features.astype(jnp.float32)
    sf = support_features.astype(jnp.float32)
    labels_i32 = support_labels.astype(jnp.int32)

    labels_pad = jnp.concatenate(
        [labels_i32, jnp.full((S_PAD - S,), NCLS, jnp.int32)]).reshape(
            NBLK, 1, SB)

    scores, cmax_raw, pscores, smx, pconf, ppred = _stage1(qf, sf,
                                                           labels_pad)
    cmax = cmax_raw.transpose(1, 0, 2).reshape(Q, NC)
    rows = _stage2(cmax)

    indices, kdist, knn_pred, new_pred, fused = _stage3_sc(
        scores.reshape(Q * NC, CH), rows, labels_i32, smx,
        pconf.reshape(Q), ppred.reshape(Q))

    return (pscores, indices, kdist,
            knn_pred.astype(support_labels.dtype),
            new_pred.astype(support_labels.dtype), fused)


NW = 32
QW = Q // NW
NG = 16


def _sc_body(scores_hbm, rows_hbm, labels_hbm, smx_hbm, pconf_hbm, ppred_hbm,
             idx_out, kd_out, kp_out, np_out, fu_out,
             labels_v, rows_v, cand_v, labsel_v, w_v,
             oidx_v, okd_v, okp_v, onp_v, ofu_v, smx_v, pconf_v, ppred_v,
             sem):
    wid = lax.axis_index("s") * 2 + lax.axis_index("c")
    q0 = (wid * QW).astype(jnp.int32)
    lane = lax.broadcasted_iota(jnp.int32, (16,), 0)
    izeros = jnp.zeros((16,), jnp.int32)
    lane0 = lane == 0

    pltpu.sync_copy(labels_hbm, labels_v)
    pltpu.sync_copy(rows_hbm.at[pl.ds(q0, QW)], rows_v)
    pltpu.sync_copy(smx_hbm.at[pl.ds(q0, QW)], smx_v)
    pltpu.sync_copy(pconf_hbm.at[pl.ds(q0, QW)], pconf_v)
    pltpu.sync_copy(ppred_hbm.at[pl.ds(q0, QW)], ppred_v)

    def gmax_one(p, g):
        gm = jnp.full((16,), NEG, jnp.float32)
        for t in range(NG):
            row = 2 * g + t // 8
            vt = cand_v[p, row, pl.ds((t % 8) * 16, 16)]
            gm = jnp.maximum(gm, vt)
        return jnp.max(gm)

    pltpu.async_copy(scores_hbm.at[rows_v.at[jnp.int32(0)]],
                     cand_v.at[jnp.int32(0)], sem)

    def per_query(_, carry):
        i, kp0, kp1, np0, np1 = carry
        q = q0 + i
        p = jnp.bitwise_and(i, jnp.int32(1))
        pltpu.make_async_copy(
            scores_hbm.at[rows_v.at[i]], cand_v.at[p], sem).wait()

        @pl.when(i < QW - 1)
        def _prefetch():
            pltpu.async_copy(
                scores_hbm.at[rows_v.at[i + 1]], cand_v.at[1 - p], sem)

        def build(_, bcarry):
            g, M = bcarry
            return g + 1, jnp.where(lane == g, gmax_one(p, g), M)

        _, M = lax.fori_loop(
            0, NG, build,
            (jnp.int32(0), jnp.full((16,), NEG, jnp.float32)))

        def ext(_, ecarry):
            k, M, sv0, sv1, iv0, iv1 = ecarry
            m = jnp.max(M)
            g = jnp.min(jnp.where(M == m, lane, NG))
            fp = jnp.int32(4096)
            for t in range(NG):
                row = 2 * g + t // 8
                vt = cand_v[p, row, pl.ds((t % 8) * 16, 16)]
                fp = jnp.minimum(
                    fp, jnp.min(jnp.where(vt == m, t * 16 + lane, 4096)))
            f = g * 256 + fp
            row_f = lax.shift_right_logical(f, jnp.int32(7))
            col_f = jnp.bitwise_and(f, jnp.int32(127))
            plsc.store_scatter(cand_v,
                               [izeros + p, izeros + row_f, izeros + col_f],
                               jnp.full((16,), NEG, jnp.float32), mask=lane0)
            M = jnp.where(lane == g, gmax_one(p, g), M)
            r_val = plsc.load_gather(
                rows_v, [izeros + i, izeros + row_f])[0]
            sup = (r_val - q * NC) * CH + col_f
            sv0 = jnp.where(lane == k, m, sv0)
            sv1 = jnp.where(lane == k - 16, m, sv1)
            iv0 = jnp.where(lane == k, sup, iv0)
            iv1 = jnp.where(lane == k - 16, sup, iv1)
            return k + 1, M, sv0, sv1, iv0, iv1

        fneg = jnp.full((16,), NEG, jnp.float32)
        _, M, sv0, sv1, iv0, iv1 = lax.fori_loop(
            0, K, ext, (jnp.int32(0), M, fneg, fneg, izeros, izeros))

        oidx_v[i, pl.ds(0, 16)] = iv0
        oidx_v[i, pl.ds(16, 16)] = iv1
        okd_v[i, pl.ds(0, 16)] = -sv0
        okd_v[i, pl.ds(16, 16)] = -sv1

        m0 = jnp.max(sv0)
        e0 = jnp.exp(sv0 - m0)
        e1 = jnp.exp(sv1 - m0)
        z = jnp.sum(e0) + jnp.sum(e1)
        w_v[pl.ds(0, 16)] = e0 / z
        w_v[pl.ds(16, 16)] = e1 / z

        labsel_v[pl.ds(0, 16)] = plsc.load_gather(labels_v, [iv0])
        labsel_v[pl.ds(16, 16)] = plsc.load_gather(labels_v, [iv1])

        cls = [lane + 16 * b for b in range(4)]

        def vote(_, vcarry):
            k = vcarry[0]
            v = list(vcarry[1:5])
            s = list(vcarry[5:])
            lab = plsc.load_gather(labsel_v, [izeros + k])[0]
            wk = plsc.load_gather(w_v, [izeros + k])[0]
            one = jnp.float32(1.0)
            zero = jnp.float32(0.0)
            for b in range(4):
                eq = cls[b] == lab
                v[b] = v[b] + jnp.where(eq, one, zero)
                s[b] = s[b] + jnp.where(eq, wk, zero)
            return (k + 1,) + tuple(v) + tuple(s)

        zeros = tuple(jnp.zeros((16,), jnp.float32) for _ in range(8))
        vs = lax.fori_loop(0, K, vote, (jnp.int32(0),) + zeros)
        votes = vs[1:5]
        soft = vs[5:]

        best_v = jnp.float32(-1.0)
        best_c = jnp.int32(NCLS)
        for b in range(4):
            bm = jnp.max(votes[b])
            pos = jnp.min(jnp.where(votes[b] == bm, cls[b], NCLS))
            take = bm > best_v
            best_c = jnp.where(take, pos, best_c)
            best_v = jnp.where(take, bm, best_v)
        knn_conf = best_v * jnp.float32(1.0 / K)
        npred = jnp.where(
            jnp.float32(LAM) * plsc.load_gather(pconf_v, [izeros + i])[0]
            > jnp.float32(1.0 - LAM) * knn_conf,
            plsc.load_gather(ppred_v, [izeros + i])[0], best_c)
        kp0 = jnp.where(lane == i, best_c, kp0)
        kp1 = jnp.where(lane == i - 16, best_c, kp1)
        np0 = jnp.where(lane == i, npred, np0)
        np1 = jnp.where(lane == i - 16, npred, np1)
        half = jnp.float32(0.5)
        for b in range(4):
            ofu_v[i, pl.ds(16 * b, 16)] = (
                half * smx_v[i, pl.ds(16 * b, 16)] + half * soft[b])
        return i + 1, kp0, kp1, np0, np1

    _, kp0, kp1, np0, np1 = lax.fori_loop(
        0, QW, per_query,
        (jnp.int32(0), izeros, izeros, izeros, izeros))
    okp_v[pl.ds(0, 16)] = kp0
    okp_v[pl.ds(16, 16)] = kp1
    onp_v[pl.ds(0, 16)] = np0
    onp_v[pl.ds(16, 16)] = np1

    pltpu.sync_copy(oidx_v, idx_out.at[pl.ds(q0, QW)])
    pltpu.sync_copy(okd_v, kd_out.at[pl.ds(q0, QW)])
    pltpu.sync_copy(okp_v, kp_out.at[pl.ds(q0, QW)])
    pltpu.sync_copy(onp_v, np_out.at[pl.ds(q0, QW)])
    pltpu.sync_copy(ofu_v, fu_out.at[pl.ds(q0, QW)])


def _stage3_sc(scores2d, rows, labels_i32, smx, pconf, ppred):
    mesh = plsc.VectorSubcoreMesh(core_axis_name="c", subcore_axis_name="s")
    f32 = jnp.float32
    i32 = jnp.int32
    sk = functools.partial(
        pl.kernel, mesh=mesh,
        compiler_params=pltpu.CompilerParams(needs_layout_passes=False),
        out_type=(
            jax.ShapeDtypeStruct((Q, K), i32),
            jax.ShapeDtypeStruct((Q, K), f32),
            jax.ShapeDtypeStruct((Q,), i32),
            jax.ShapeDtypeStruct((Q,), i32),
            jax.ShapeDtypeStruct((Q, NCLS), f32),
        ),
        scratch_types=[
            pltpu.VMEM((S,), i32),
            pltpu.VMEM((QW, K), i32),
            pltpu.VMEM((2, K, CH), f32),
            pltpu.VMEM((K,), i32),
            pltpu.VMEM((K,), f32),
            pltpu.VMEM((QW, K), i32),
            pltpu.VMEM((QW, K), f32),
            pltpu.VMEM((QW,), i32),
            pltpu.VMEM((QW,), i32),
            pltpu.VMEM((QW, NCLS), f32),
            pltpu.VMEM((QW, NCLS), f32),
            pltpu.VMEM((QW,), f32),
            pltpu.VMEM((QW,), i32),
            pltpu.SemaphoreType.DMA,
        ],
    )(_sc_body)
    return sk(scores2d, rows, labels_i32, smx, pconf, ppred)


# speedup vs baseline: 9.8000x; 1.0006x over previous
import functools

import jax
import jax.numpy as jnp
from jax import lax
from jax.experimental import pallas as pl
from jax.experimental.pallas import tpu as pltpu
from jax.experimental.pallas import tpu_sc as plsc

Q = 1024
S = 100000
D = 256
NCLS = 64
K = 32
LAM = 0.5
CH = 128
SB = 2048
S_PAD = 100352
NBLK = S_PAD // SB
NC = S_PAD // CH
QT = 128
NEG = -3.0e38


def _i0():
    return jnp.int32(0)


def _stage1_body(q_ref, s_ref, l_ref, scores_ref, cmax_ref, ps_ref, smx_ref,
                 pconf_ref, ppred_ref, acc_ref, cnt_ref, qsq_ref, x2_ref):
    b = pl.program_id(0)
    x = q_ref[...]
    sblk = s_ref[...]
    labels = l_ref[0, 0, :]

    cls = lax.broadcasted_iota(jnp.int32, (NCLS, SB), 0)
    onehot_t = (labels[None, :] == cls).astype(jnp.float32)
    pacc = lax.dot_general(onehot_t, sblk, (((1,), (0,)), ((), ())),
                           preferred_element_type=jnp.float32)

    @pl.when(b == 0)
    def _init():
        acc_ref[...] = jnp.zeros_like(acc_ref)
        cnt_ref[...] = jnp.zeros_like(cnt_ref)
        qsq_ref[...] = jnp.sum(x * x, axis=1, keepdims=True)
        x2_ref[...] = x + x

    acc_ref[...] += pacc
    cnt_ref[...] += jnp.sum(onehot_t, axis=1, keepdims=True)

    q_sq = qsq_ref[...]
    s_sq = jnp.sum(sblk * sblk, axis=1)[None, :]
    qs2 = lax.dot_general(x2_ref[...], sblk, (((1,), (1,)), ((), ())),
                          preferred_element_type=jnp.float32)
    scores = (qs2 - q_sq) - s_sq

    @pl.when(b < NBLK - 1)
    def _store_full():
        scores_ref[...] = scores
        cmax_ref[0] = jnp.max(scores.reshape(Q, SB // CH, CH), axis=2)

    @pl.when(b == NBLK - 1)
    def _store_masked():
        col = lax.broadcasted_iota(jnp.int32, (Q, SB), 1) + b * SB
        sm = jnp.where(col < S, scores, jnp.float32(NEG))
        scores_ref[...] = sm
        cmax_ref[0] = jnp.max(sm.reshape(Q, SB // CH, CH), axis=2)

    @pl.when(b == NBLK - 1)
    def _final():
        counts = jnp.maximum(cnt_ref[...], jnp.float32(1.0))
        protos = acc_ref[...] / counts
        p_sq = jnp.sum(protos * protos, axis=1)[None, :]
        qp = lax.dot_general(x, protos, (((1,), (1,)), ((), ())),
                             preferred_element_type=jnp.float32)
        pscores = -((q_sq - jnp.float32(2.0) * qp) + p_sq)
        ps_ref[...] = pscores
        m = jnp.max(pscores, axis=1, keepdims=True)
        e = jnp.exp(pscores - m)
        z = jnp.sum(e, axis=1, keepdims=True)
        smx = e / z
        smx_ref[...] = smx
        pconf_ref[...] = jnp.max(smx, axis=1, keepdims=True)
        ci = lax.broadcasted_iota(jnp.int32, (Q, NCLS), 1)
        ppred_ref[...] = jnp.min(
            jnp.where(pscores == m, ci, NCLS), axis=1, keepdims=True)


def _stage1(qf, sf, labels_pad):
    out_shapes = (
        jax.ShapeDtypeStruct((Q, S_PAD), jnp.float32),
        jax.ShapeDtypeStruct((NBLK, Q, SB // CH), jnp.float32),
        jax.ShapeDtypeStruct((Q, NCLS), jnp.float32),
        jax.ShapeDtypeStruct((Q, NCLS), jnp.float32),
        jax.ShapeDtypeStruct((Q, 1), jnp.float32),
        jax.ShapeDtypeStruct((Q, 1), jnp.int32),
    )
    return pl.pallas_call(
        _stage1_body,
        grid=(NBLK,),
        in_specs=[
            pl.BlockSpec((Q, D), lambda b: (_i0(), _i0())),
            pl.BlockSpec((SB, D), lambda b: (b, _i0())),
            pl.BlockSpec((1, 1, SB), lambda b: (b, _i0(), _i0())),
        ],
        out_specs=(
            pl.BlockSpec((Q, SB), lambda b: (_i0(), b)),
            pl.BlockSpec((1, Q, SB // CH), lambda b: (b, _i0(), _i0())),
            pl.BlockSpec((Q, NCLS), lambda b: (_i0(), _i0())),
            pl.BlockSpec((Q, NCLS), lambda b: (_i0(), _i0())),
            pl.BlockSpec((Q, 1), lambda b: (_i0(), _i0())),
            pl.BlockSpec((Q, 1), lambda b: (_i0(), _i0())),
        ),
        out_shape=out_shapes,
        scratch_shapes=[
            pltpu.VMEM((NCLS, D), jnp.float32),
            pltpu.VMEM((NCLS, 1), jnp.float32),
            pltpu.VMEM((Q, 1), jnp.float32),
            pltpu.VMEM((Q, D), jnp.float32),
        ],
    )(qf, sf, labels_pad)


def _stage2_body(cm_ref, rows_ref, x_ref, cand_ref):
    t = pl.program_id(0)
    ci = lax.broadcasted_iota(jnp.int32, (QT, NC), 1)
    x_ref[...] = cm_ref[...]
    ki = lax.broadcasted_iota(jnp.int32, (QT, K), 1)

    def pick(_, c):
        xx = x_ref[...]
        m = jnp.max(xx, axis=1, keepdims=True)
        cid = jnp.min(jnp.where(xx == m, ci, NC), axis=1, keepdims=True)
        x_ref[...] = jnp.where(ci == cid, jnp.float32(NEG), xx)
        cand_ref[...] = jnp.where(ki == c, cid, cand_ref[...])
        return c + 1

    lax.fori_loop(0, K, pick, jnp.int32(0), unroll=False)

    qi = lax.broadcasted_iota(jnp.int32, (QT, K), 0) + t * QT

    def srt(_, c):
        cc = cand_ref[...]
        mn = jnp.min(cc, axis=1, keepdims=True)
        cand_ref[...] = jnp.where(cc == mn, NC + 1, cc)
        rows_ref[...] = jnp.where(ki == c, qi * NC + mn, rows_ref[...])
        return c + 1

    lax.fori_loop(0, K, srt, jnp.int32(0), unroll=False)


def _stage2(cmax):
    return pl.pallas_call(
        _stage2_body,
        grid=(Q // QT,),
        in_specs=[pl.BlockSpec((QT, NC), lambda t: (t, _i0()))],
        out_specs=pl.BlockSpec((QT, K), lambda t: (t, _i0())),
        out_shape=jax.ShapeDtypeStruct((Q, K), jnp.int32),
        scratch_shapes=[
            pltpu.VMEM((QT, NC), jnp.float32),
            pltpu.VMEM((QT, K), jnp.int32),
        ],
    )(cmax)


def kernel(query_features, support_features, support_labels):
    qf = query_features.astype(jnp.float32)
    sf = support_features.astype(jnp.float32)
    labels_i32 = support_labels.astype(jnp.int32)

    labels_pad = jnp.concatenate(
        [labels_i32, jnp.full((S_PAD - S,), NCLS, jnp.int32)]).reshape(
            NBLK, 1, SB)

    scores, cmax_raw, pscores, smx, pconf, ppred = _stage1(qf, sf,
                                                           labels_pad)
    cmax = cmax_raw.transpose(1, 0, 2).reshape(Q, NC)
    rows = _stage2(cmax)

    indices_f, kdist_f, knn_pred, new_pred, fused_f = _stage3_sc(
        scores.reshape(Q * NC, CH), rows.reshape(Q * K), labels_i32,
        smx.reshape(Q * NCLS), pconf.reshape(Q), ppred.reshape(Q))
    indices = indices_f.reshape(Q, K)
    kdist = kdist_f.reshape(Q, K)
    fused = fused_f.reshape(Q, NCLS)

    return (pscores, indices, kdist,
            knn_pred.astype(support_labels.dtype),
            new_pred.astype(support_labels.dtype), fused)


NW = 32
QW = Q // NW
NG = 16


def _sc_body(scores_hbm, rows_hbm, labels_hbm, smx_hbm, pconf_hbm, ppred_hbm,
             idx_out, kd_out, kp_out, np_out, fu_out,
             labels_v, rows_v, cand_v, labsel_v, w_v, tb_v,
             oidx_v, okd_v, okp_v, onp_v, ofu_v, smx_v, pconf_v, ppred_v,
             sem):
    wid = lax.axis_index("s") * 2 + lax.axis_index("c")
    q0 = (wid * QW).astype(jnp.int32)
    lane = lax.broadcasted_iota(jnp.int32, (16,), 0)
    izeros = jnp.zeros((16,), jnp.int32)
    lane0 = lane == 0

    pltpu.sync_copy(labels_hbm, labels_v)
    pltpu.sync_copy(rows_hbm.at[pl.ds(q0 * K, QW * K)], rows_v)
    pltpu.sync_copy(smx_hbm.at[pl.ds(q0 * NCLS, QW * NCLS)], smx_v)
    pltpu.sync_copy(pconf_hbm.at[pl.ds(q0, QW)], pconf_v)
    pltpu.sync_copy(ppred_hbm.at[pl.ds(q0, QW)], ppred_v)

    pltpu.async_copy(scores_hbm.at[rows_v.at[pl.ds(0, K)]],
                     cand_v.at[jnp.int32(0)], sem)

    def per_query(_, carry):
        i, kp0, kp1, np0, np1 = carry
        q = q0 + i
        p = jnp.bitwise_and(i, jnp.int32(1))
        pltpu.make_async_copy(
            scores_hbm.at[rows_v.at[pl.ds(i * K, K)]],
            cand_v.at[p], sem).wait()

        @pl.when(i < QW - 1)
        def _prefetch():
            pltpu.async_copy(
                scores_hbm.at[rows_v.at[pl.ds((i + 1) * K, K)]],
                cand_v.at[1 - p], sem)

        def build(_, g):
            gm = jnp.full((16,), NEG, jnp.float32)
            for t in range(NG):
                row = 2 * g + t // 8
                gm = jnp.maximum(gm, cand_v[p, row, pl.ds((t % 8) * 16, 16)])
            tb_v[pl.ds(g * 16, 16)] = gm
            return g + 1

        lax.fori_loop(0, NG, build, jnp.int32(0))
        M = jnp.full((16,), NEG, jnp.float32)
        for j in range(NG):
            M = jnp.maximum(M, plsc.load_gather(tb_v, [lane * 16 + j]))

        def ext(_, ecarry):
            k, M, sv0, sv1, iv0, iv1 = ecarry
            m = jnp.max(M)
            g = jnp.min(jnp.where(M == m, lane, NG))
            posv = jnp.full((16,), 4096, jnp.int32)
            for t in range(NG):
                row = 2 * g + t // 8
                vt = cand_v[p, row, pl.ds((t % 8) * 16, 16)]
                posv = jnp.minimum(
                    posv, jnp.where(vt == m, t * 16 + lane, 4096))
            fp = jnp.min(posv)
            f = g * 256 + fp
            row_f = lax.shift_right_logical(f, jnp.int32(7))
            col_f = jnp.bitwise_and(f, jnp.int32(127))
            plsc.store_scatter(cand_v,
                               [izeros + p, izeros + row_f, izeros + col_f],
                               jnp.full((16,), NEG, jnp.float32), mask=lane0)
            gmv = jnp.full((16,), NEG, jnp.float32)
            for t in range(NG):
                row = 2 * g + t // 8
                gmv = jnp.maximum(gmv, cand_v[p, row, pl.ds((t % 8) * 16, 16)])
            M = jnp.where(lane == g, jnp.max(gmv), M)
            r_val = plsc.load_gather(rows_v, [izeros + (i * K + row_f)])[0]
            sup = (r_val - q * NC) * CH + col_f
            sv0 = jnp.where(lane == k, m, sv0)
            sv1 = jnp.where(lane == k - 16, m, sv1)
            iv0 = jnp.where(lane == k, sup, iv0)
            iv1 = jnp.where(lane == k - 16, sup, iv1)
            return k + 1, M, sv0, sv1, iv0, iv1

        fneg = jnp.full((16,), NEG, jnp.float32)
        _, M, sv0, sv1, iv0, iv1 = lax.fori_loop(
            0, K, ext, (jnp.int32(0), M, fneg, fneg, izeros, izeros))

        oidx_v[pl.ds(i * K, 16)] = iv0
        oidx_v[pl.ds(i * K + 16, 16)] = iv1
        okd_v[pl.ds(i * K, 16)] = -sv0
        okd_v[pl.ds(i * K + 16, 16)] = -sv1

        m0 = jnp.max(sv0)
        e0 = jnp.exp(sv0 - m0)
        e1 = jnp.exp(sv1 - m0)
        z = jnp.sum(e0) + jnp.sum(e1)
        w_v[pl.ds(0, 16)] = e0 / z
        w_v[pl.ds(16, 16)] = e1 / z

        labsel_v[pl.ds(0, 16)] = plsc.load_gather(labels_v, [iv0])
        labsel_v[pl.ds(16, 16)] = plsc.load_gather(labels_v, [iv1])

        cls = [lane + 16 * b for b in range(4)]

        def vote(_, vcarry):
            k = vcarry[0]
            v = list(vcarry[1:5])
            s = list(vcarry[5:])
            lab = plsc.load_gather(labsel_v, [izeros + k])[0]
            wk = plsc.load_gather(w_v, [izeros + k])[0]
            one = jnp.float32(1.0)
            zero = jnp.float32(0.0)
            for b in range(4):
                eq = cls[b] == lab
                v[b] = v[b] + jnp.where(eq, one, zero)
                s[b] = s[b] + jnp.where(eq, wk, zero)
            return (k + 1,) + tuple(v) + tuple(s)

        zeros = tuple(jnp.zeros((16,), jnp.float32) for _ in range(8))
        vs = lax.fori_loop(0, K, vote, (jnp.int32(0),) + zeros)
        votes = vs[1:5]
        soft = vs[5:]

        best_v = jnp.float32(-1.0)
        best_c = jnp.int32(NCLS)
        for b in range(4):
            bm = jnp.max(votes[b])
            pos = jnp.min(jnp.where(votes[b] == bm, cls[b], NCLS))
            take = bm > best_v
            best_c = jnp.where(take, pos, best_c)
            best_v = jnp.where(take, bm, best_v)
        knn_conf = best_v * jnp.float32(1.0 / K)
        npred = jnp.where(
            jnp.float32(LAM) * plsc.load_gather(pconf_v, [izeros + i])[0]
            > jnp.float32(1.0 - LAM) * knn_conf,
            plsc.load_gather(ppred_v, [izeros + i])[0], best_c)
        kp0 = jnp.where(lane == i, best_c, kp0)
        kp1 = jnp.where(lane == i - 16, best_c, kp1)
        np0 = jnp.where(lane == i, npred, np0)
        np1 = jnp.where(lane == i - 16, npred, np1)
        half = jnp.float32(0.5)
        for b in range(4):
            ofu_v[pl.ds(i * NCLS + 16 * b, 16)] = (
                half * smx_v[pl.ds(i * NCLS + 16 * b, 16)] + half * soft[b])
        return i + 1, kp0, kp1, np0, np1

    _, kp0, kp1, np0, np1 = lax.fori_loop(
        0, QW, per_query,
        (jnp.int32(0), izeros, izeros, izeros, izeros))
    okp_v[pl.ds(0, 16)] = kp0
    okp_v[pl.ds(16, 16)] = kp1
    onp_v[pl.ds(0, 16)] = np0
    onp_v[pl.ds(16, 16)] = np1

    pltpu.sync_copy(oidx_v, idx_out.at[pl.ds(q0 * K, QW * K)])
    pltpu.sync_copy(okd_v, kd_out.at[pl.ds(q0 * K, QW * K)])
    pltpu.sync_copy(okp_v, kp_out.at[pl.ds(q0, QW)])
    pltpu.sync_copy(onp_v, np_out.at[pl.ds(q0, QW)])
    pltpu.sync_copy(ofu_v, fu_out.at[pl.ds(q0 * NCLS, QW * NCLS)])


def _stage3_sc(scores2d, rows, labels_i32, smx, pconf, ppred):
    mesh = plsc.VectorSubcoreMesh(core_axis_name="c", subcore_axis_name="s")
    f32 = jnp.float32
    i32 = jnp.int32
    sk = functools.partial(
        pl.kernel, mesh=mesh,
        compiler_params=pltpu.CompilerParams(needs_layout_passes=False),
        out_type=(
            jax.ShapeDtypeStruct((Q * K,), i32),
            jax.ShapeDtypeStruct((Q * K,), f32),
            jax.ShapeDtypeStruct((Q,), i32),
            jax.ShapeDtypeStruct((Q,), i32),
            jax.ShapeDtypeStruct((Q * NCLS,), f32),
        ),
        scratch_types=[
            pltpu.VMEM((S,), i32),
            pltpu.VMEM((QW * K,), i32),
            pltpu.VMEM((2, K, CH), f32),
            pltpu.VMEM((K,), i32),
            pltpu.VMEM((K,), f32),
            pltpu.VMEM((NG * 16,), f32),
            pltpu.VMEM((QW * K,), i32),
            pltpu.VMEM((QW * K,), f32),
            pltpu.VMEM((QW,), i32),
            pltpu.VMEM((QW,), i32),
            pltpu.VMEM((QW * NCLS,), f32),
            pltpu.VMEM((QW * NCLS,), f32),
            pltpu.VMEM((QW,), f32),
            pltpu.VMEM((QW,), i32),
            pltpu.SemaphoreType.DMA,
        ],
    )(_sc_body)
    return sk(scores2d, rows, labels_i32, smx, pconf, ppred)
